# baseline (speedup 1.0000x reference)
import functools

import jax
import jax.numpy as jnp
from jax import lax
from jax.experimental import pallas as pl
from jax.experimental.pallas import tpu as pltpu
from jax.experimental.pallas import tpu_sc as plsc

N = 10000
D = 128
E = 320000
H1, C1 = 8, 8
NCLS = 8

NP = 10240
DUMMY = N
NC, NS = 2, 16
NTILES = NC * NS
CH = 64
CPT = 168
NBUF = 4
ZR = 32
EPAD = NTILES * CPT * CH
RW1 = 80
RW2 = 16
BLK = 2048
ROWS_PER_TILE = NP // NS


def _leaky(x):
  return jnp.maximum(x, 0.2 * x)


def _vgather(v, idx):
  dnums = lax.GatherDimensionNumbers(
      offset_dims=(), collapsed_slice_dims=(0,), start_index_map=(0,))
  return lax.gather(v, idx[:, None], dnums, slice_sizes=(1,),
                    mode=lax.GatherScatterMode.PROMISE_IN_BOUNDS)


def _stage_a_body(x_ref, w_ref, as_ref, ad_ref, t1_ref, ad1_ref, ms_ref,
                  md_ref):
  i = pl.program_id(0)
  h = jnp.dot(x_ref[...], w_ref[...], preferred_element_type=jnp.float32)
  rr = lax.broadcasted_iota(jnp.int32, (H1 * C1, H1), 0)
  cc = lax.broadcasted_iota(jnp.int32, (H1 * C1, H1), 1)
  g = (rr // C1 == cc).astype(jnp.float32)
  a_s = jnp.dot(h * as_ref[...], g, preferred_element_type=jnp.float32)
  a_d = jnp.dot(h * ad_ref[...], g, preferred_element_type=jnp.float32)
  zpad = jnp.zeros((h.shape[0], 8), jnp.float32)
  t1_ref[...] = jnp.concatenate([a_s, zpad, h], axis=1)
  ad1_ref[...] = jnp.concatenate([a_d, zpad], axis=1)
  cs = jnp.max(a_s, axis=0, keepdims=True)
  cd = jnp.max(a_d, axis=0, keepdims=True)

  @pl.when(i == 0)
  def _():
    ms_ref[...] = cs
    md_ref[...] = cd

  @pl.when(i > 0)
  def _():
    ms_ref[...] = jnp.maximum(ms_ref[...], cs)
    md_ref[...] = jnp.maximum(md_ref[...], cd)


def _stage_a(x_pad, w1, as1, ad1):
  return pl.pallas_call(
      _stage_a_body,
      grid=(NP // BLK,),
      in_specs=[
          pl.BlockSpec((BLK, D), lambda i: (i, 0)),
          pl.BlockSpec((D, H1 * C1), lambda i: (0, 0)),
          pl.BlockSpec((1, H1 * C1), lambda i: (0, 0)),
          pl.BlockSpec((1, H1 * C1), lambda i: (0, 0)),
      ],
      out_specs=[
          pl.BlockSpec((BLK, RW1), lambda i: (i, 0)),
          pl.BlockSpec((BLK, 16), lambda i: (i, 0)),
          pl.BlockSpec((1, 8), lambda i: (0, 0)),
          pl.BlockSpec((1, 8), lambda i: (0, 0)),
      ],
      out_shape=[
          jax.ShapeDtypeStruct((NP, RW1), jnp.float32),
          jax.ShapeDtypeStruct((NP, 16), jnp.float32),
          jax.ShapeDtypeStruct((1, 8), jnp.float32),
          jax.ShapeDtypeStruct((1, 8), jnp.float32),
      ],
  )(x_pad, w1, as1, ad1)


def _ivec(v):
  return jnp.full((16,), v, jnp.int32)


def _make_edge_fn_l1(mv):
  base = lax.div(lax.broadcasted_iota(jnp.int32, (16,), 0), _ivec(8))
  idxs = [lax.add(base, _ivec(2 * k)) for k in range(4)]

  def edge(rows, adr, out, i):
    va = rows[i, pl.ds(0, 16)]
    vd = adr[i, pl.ds(0, 16)]
    al = va + vd
    e = jnp.exp(_leaky(al) - mv)
    out[i, pl.ds(0, 16)] = e
    for k in range(4):
      hk = rows[i, pl.ds(16 + 16 * k, 16)]
      mk = _vgather(e, idxs[k])
      out[i, pl.ds(16 + 16 * k, 16)] = hk * mk
  return edge


def _make_edge_fn_l2(mv):
  lane = lax.broadcasted_iota(jnp.int32, (16,), 0)
  eight = _ivec(8)
  msk = lane < eight

  def edge(rows, adr, out, i):
    r = rows[i, pl.ds(0, 16)]
    d = adr[i, pl.ds(0, 16)]
    al = r + d
    ef = jnp.exp(_leaky(al) - mv)
    g = _vgather(ef, eight)
    out[i, pl.ds(0, 16)] = jnp.where(msk, g * r, g)
  return edge


def _sc_edge_pass(rw, make_edge_fn, nbuf, inplace):
  mesh = plsc.VectorSubcoreMesh(core_axis_name="c", subcore_axis_name="s")

  def body(t_hbm, ad_hbm, src_hbm, dst_hbm, m_hbm, out_hbm,
           idx_s, idx_d, rows, adr, mvec, accum,
           sg_r, sg_a, ss, outb=None):
    c = lax.axis_index("c")
    s = lax.axis_index("s")
    wid = s * NC + c
    pltpu.sync_copy(src_hbm.at[wid], idx_s)
    pltpu.sync_copy(dst_hbm.at[wid], idx_d)
    pltpu.sync_copy(m_hbm, mvec)
    edge_fn = make_edge_fn(mvec[...])

    zb = rows.at[0]

    def zrow(r, carry):
      for k in range(rw // 16):
        zb[r, pl.ds(16 * k, 16)] = jnp.zeros((16,), jnp.float32)
      return carry
    lax.fori_loop(0, CH, zrow, 0)
    nfull, remr = ROWS_PER_TILE // CH, ROWS_PER_TILE % CH
    for k in range(nfull):
      pltpu.sync_copy(zb, accum.at[pl.ds(s * ROWS_PER_TILE + k * CH, CH)])
    if remr:
      pltpu.sync_copy(
          rows.at[0, pl.ds(0, remr)],
          accum.at[pl.ds(s * ROWS_PER_TILE + nfull * CH, remr)])
    plsc.subcore_barrier()

    def start_gathers(j, b):
      pltpu.async_copy(t_hbm.at[idx_s.at[j]], rows.at[b], sg_r.at[b])
      pltpu.async_copy(ad_hbm.at[idx_d.at[j]], adr.at[b], sg_a.at[b])

    def wait_gathers(j, b):
      pltpu.make_async_copy(t_hbm.at[idx_s.at[j]], rows.at[b],
                            sg_r.at[b]).wait()
      pltpu.make_async_copy(ad_hbm.at[idx_d.at[j]], adr.at[b],
                            sg_a.at[b]).wait()

    if inplace:
      def wait_scatter(j, b):
        pltpu.make_async_copy(rows.at[b], accum.at[idx_d.at[j]],
                              ss.at[b]).wait()

      for b in range(nbuf - 1):
        start_gathers(b, b)

      def outer(g, carry):
        for b in range(nbuf):
          j = g * nbuf + b
          wait_gathers(j, b)

          def edge2(i, c2):
            edge_fn(rows.at[b], adr.at[b], rows.at[b], i * 2)
            edge_fn(rows.at[b], adr.at[b], rows.at[b], i * 2 + 1)
            return c2
          lax.fori_loop(0, CH // 2, edge2, 0)
          pltpu.async_copy(rows.at[b], accum.at[idx_d.at[j]], ss.at[b],
                           add=True)
          bp = (b - 1) % nbuf

          @pl.when(j >= 1)
          def _():
            wait_scatter(j - 1, bp)

          @pl.when(j + nbuf - 1 < CPT)
          def _():
            start_gathers(j + nbuf - 1, bp)
        return carry
      lax.fori_loop(0, CPT // nbuf, outer, 0)
      wait_scatter(CPT - 1, (CPT - 1) % nbuf)
    else:
      def wait_scatter(j, b):
        pltpu.make_async_copy(outb.at[b], accum.at[idx_d.at[j]],
                              ss.at[b]).wait()

      for b in range(nbuf):
        start_gathers(b, b)

      def outer(g, carry):
        for b in range(nbuf):
          j = g * nbuf + b
          wait_gathers(j, b)
          ob = j % 2

          @pl.when(j >= 2)
          def _():
            wait_scatter(j - 2, ob)

          def edge2(i, c2):
            edge_fn(rows.at[b], adr.at[b], outb.at[ob], i * 2)
            edge_fn(rows.at[b], adr.at[b], outb.at[ob], i * 2 + 1)
            return c2
          lax.fori_loop(0, CH // 2, edge2, 0)
          pltpu.async_copy(outb.at[ob], accum.at[idx_d.at[j]], ss.at[ob],
                           add=True)

          @pl.when(j + nbuf < CPT)
          def _():
            start_gathers(j + nbuf, b)
        return carry
      lax.fori_loop(0, CPT // nbuf, outer, 0)
      wait_scatter(CPT - 2, (CPT - 2) % 2)
      wait_scatter(CPT - 1, (CPT - 1) % 2)
    plsc.subcore_barrier()
    for k in range(nfull):
      pltpu.sync_copy(accum.at[pl.ds(s * ROWS_PER_TILE + k * CH, CH)],
                      rows.at[0])
      pltpu.sync_copy(rows.at[0],
                      out_hbm.at[c, pl.ds(s * ROWS_PER_TILE + k * CH, CH)])
    if remr:
      off = s * ROWS_PER_TILE + nfull * CH
      pltpu.sync_copy(accum.at[pl.ds(off, remr)], rows.at[0, pl.ds(0, remr)])
      pltpu.sync_copy(rows.at[0, pl.ds(0, remr)],
                      out_hbm.at[c, pl.ds(off, remr)])

  scratch = [
      pltpu.VMEM((CPT, CH), jnp.int32),
      pltpu.VMEM((CPT, CH), jnp.int32),
      pltpu.VMEM((nbuf, CH, rw), jnp.float32),
      pltpu.VMEM((nbuf, CH, 16), jnp.float32),
      pltpu.VMEM((16,), jnp.float32),
      pltpu.VMEM_SHARED((NP, rw), jnp.float32),
      pltpu.SemaphoreType.DMA((nbuf,)),
      pltpu.SemaphoreType.DMA((nbuf,)),
      pltpu.SemaphoreType.DMA((nbuf,)),
  ]
  if not inplace:
    scratch.append(pltpu.VMEM((2, CH, rw), jnp.float32))
  kern = pl.kernel(
      body,
      out_type=jax.ShapeDtypeStruct((NC, NP, rw), jnp.float32),
      mesh=mesh,
      scratch_types=scratch,
      compiler_params=pltpu.CompilerParams(use_tc_tiling_on_sc=False),
  )
  return kern


def _stage_b_body(acc_ref, b1_ref, w2_ref, as2_ref, ad2_ref,
                  t2_ref, ad2o_ref, ms_ref, md_ref):
  i = pl.program_id(0)
  tot = acc_ref[0] + acc_ref[1]
  den = tot[:, 0:8]
  num = tot[:, 16:80]
  recip = 1.0 / (den + 1e-16)
  rr = lax.broadcasted_iota(jnp.int32, (H1, H1 * C1), 0)
  cc = lax.broadcasted_iota(jnp.int32, (H1, H1 * C1), 1)
  e8 = (cc // C1 == rr).astype(jnp.float32)
  out1 = num * jnp.dot(recip, e8, preferred_element_type=jnp.float32)
  out1 = out1 + b1_ref[...]
  x2 = jnp.where(out1 > 0, out1, jnp.exp(out1) - 1.0)
  h2 = jnp.dot(x2, w2_ref[...], preferred_element_type=jnp.float32)
  a_s2 = jnp.sum(h2 * as2_ref[...], axis=1, keepdims=True)
  a_d2 = jnp.sum(h2 * ad2_ref[...], axis=1, keepdims=True)
  blk = h2.shape[0]
  t2_ref[...] = jnp.concatenate(
      [h2, jnp.broadcast_to(a_s2, (blk, 8))], axis=1)
  ad2o_ref[...] = jnp.broadcast_to(a_d2, (blk, 16))
  cs = jnp.max(a_s2, axis=0, keepdims=True)
  cd = jnp.max(a_d2, axis=0, keepdims=True)

  @pl.when(i == 0)
  def _():
    ms_ref[...] = cs
    md_ref[...] = cd

  @pl.when(i > 0)
  def _():
    ms_ref[...] = jnp.maximum(ms_ref[...], cs)
    md_ref[...] = jnp.maximum(md_ref[...], cd)


def _stage_b(acc1, b1, w2, as2, ad2):
  return pl.pallas_call(
      _stage_b_body,
      grid=(NP // BLK,),
      in_specs=[
          pl.BlockSpec((NC, BLK, RW1), lambda i: (0, i, 0)),
          pl.BlockSpec((1, H1 * C1), lambda i: (0, 0)),
          pl.BlockSpec((H1 * C1, NCLS), lambda i: (0, 0)),
          pl.BlockSpec((1, NCLS), lambda i: (0, 0)),
          pl.BlockSpec((1, NCLS), lambda i: (0, 0)),
      ],
      out_specs=[
          pl.BlockSpec((BLK, RW2), lambda i: (i, 0)),
          pl.BlockSpec((BLK, 16), lambda i: (i, 0)),
          pl.BlockSpec((1, 1), lambda i: (0, 0)),
          pl.BlockSpec((1, 1), lambda i: (0, 0)),
      ],
      out_shape=[
          jax.ShapeDtypeStruct((NP, RW2), jnp.float32),
          jax.ShapeDtypeStruct((NP, 16), jnp.float32),
          jax.ShapeDtypeStruct((1, 1), jnp.float32),
          jax.ShapeDtypeStruct((1, 1), jnp.float32),
      ],
  )(acc1, b1, w2, as2, ad2)


def _stage_c_body(acc_ref, b2_ref, out_ref):
  tot = acc_ref[0] + acc_ref[1]
  num = tot[:, 0:8]
  den = tot[:, 8:9]
  o = num * (1.0 / (den + 1e-16)) + b2_ref[...]
  m = jnp.max(o, axis=1, keepdims=True)
  z = o - m
  lse = jnp.log(jnp.sum(jnp.exp(z), axis=1, keepdims=True))
  out_ref[...] = z - lse


def _stage_c(acc2, b2):
  return pl.pallas_call(
      _stage_c_body,
      grid=(NP // BLK,),
      in_specs=[
          pl.BlockSpec((NC, BLK, RW2), lambda i: (0, i, 0)),
          pl.BlockSpec((1, NCLS), lambda i: (0, 0)),
      ],
      out_specs=pl.BlockSpec((BLK, NCLS), lambda i: (i, 0)),
      out_shape=jax.ShapeDtypeStruct((NP, NCLS), jnp.float32),
  )(acc2, b2)


def kernel(x, edge_index, W1, att_src1, att_dst1, b1, W2, att_src2, att_dst2,
           b2):
  f32 = jnp.float32
  x_pad = jnp.pad(x, ((0, NP - N), (0, 0)))
  loop = jnp.arange(N, dtype=jnp.int32)
  npad = EPAD - E - N
  padidx = DUMMY + jnp.arange(npad, dtype=jnp.int32) % (NP - N)
  src = jnp.concatenate([edge_index[0].astype(jnp.int32), loop, padidx])
  dst = jnp.concatenate([edge_index[1].astype(jnp.int32), loop, padidx])
  src_r = src.reshape(NTILES, CPT, CH)
  dst_r = dst.reshape(NTILES, CPT, CH)

  as1 = att_src1.reshape(1, H1 * C1).astype(f32)
  ad1 = att_dst1.reshape(1, H1 * C1).astype(f32)
  t1, ad1t, ms1, md1 = _stage_a(x_pad, W1.astype(f32), as1, ad1)
  m1 = _leaky(ms1 + md1).reshape(H1)
  m1vec = jnp.concatenate([m1, jnp.zeros((8,), f32)])

  acc1 = _sc_edge_pass(RW1, _make_edge_fn_l1, 8, True)(
      t1, ad1t, src_r, dst_r, m1vec)

  as2 = att_src2.reshape(1, NCLS).astype(f32)
  ad2 = att_dst2.reshape(1, NCLS).astype(f32)
  t2, ad2t, ms2, md2 = _stage_b(acc1, b1.reshape(1, H1 * C1).astype(f32),
                                W2.astype(f32), as2, ad2)
  m2 = _leaky(ms2 + md2).reshape(())
  m2vec = jnp.full((16,), m2, f32)

  acc2 = _sc_edge_pass(RW2, _make_edge_fn_l2, 8, False)(
      t2, ad2t, src_r, dst_r, m2vec)

  out = _stage_c(acc2, b2.reshape(1, NCLS).astype(f32))
  return out[:N]


# speedup vs baseline: 1.0476x; 1.0476x over previous
import jax
import jax.numpy as jnp
from jax import lax
from jax.experimental import pallas as pl
from jax.experimental.pallas import tpu as pltpu
from jax.experimental.pallas import tpu_sc as plsc

N = 10000
D = 128
E = 320000
H1, C1 = 8, 8
NCLS = 8

NP = 10240
DUMMY = N
NC, NS = 2, 16
NTILES = NC * NS
CH = 96
CPT = 108
EPAD = NTILES * CPT * CH
RW1 = 80
RW2 = 16
BLK = 2048
ROWS_PER_TILE = NP // NS


def _leaky(x):
  return jnp.maximum(x, 0.2 * x)


def _vgather(v, idx):
  dnums = lax.GatherDimensionNumbers(
      offset_dims=(), collapsed_slice_dims=(0,), start_index_map=(0,))
  return lax.gather(v, idx[:, None], dnums, slice_sizes=(1,),
                    mode=lax.GatherScatterMode.PROMISE_IN_BOUNDS)


def _stage_a_body(x_ref, w_ref, as_ref, ad_ref, t1_ref, ad1_ref, ms_ref,
                  md_ref):
  i = pl.program_id(0)
  h = jnp.dot(x_ref[...], w_ref[...], preferred_element_type=jnp.float32)
  rr = lax.broadcasted_iota(jnp.int32, (H1 * C1, H1), 0)
  cc = lax.broadcasted_iota(jnp.int32, (H1 * C1, H1), 1)
  g = (rr // C1 == cc).astype(jnp.float32)
  a_s = jnp.dot(h * as_ref[...], g, preferred_element_type=jnp.float32)
  a_d = jnp.dot(h * ad_ref[...], g, preferred_element_type=jnp.float32)
  zpad = jnp.zeros((h.shape[0], 8), jnp.float32)
  t1_ref[...] = jnp.concatenate([a_s, zpad, h], axis=1)
  ad1_ref[...] = jnp.concatenate([a_d, zpad], axis=1)
  cs = jnp.max(a_s, axis=0, keepdims=True)
  cd = jnp.max(a_d, axis=0, keepdims=True)

  @pl.when(i == 0)
  def _():
    ms_ref[...] = cs
    md_ref[...] = cd

  @pl.when(i > 0)
  def _():
    ms_ref[...] = jnp.maximum(ms_ref[...], cs)
    md_ref[...] = jnp.maximum(md_ref[...], cd)


def _stage_a(x_pad, w1, as1, ad1):
  return pl.pallas_call(
      _stage_a_body,
      grid=(NP // BLK,),
      in_specs=[
          pl.BlockSpec((BLK, D), lambda i: (i, 0)),
          pl.BlockSpec((D, H1 * C1), lambda i: (0, 0)),
          pl.BlockSpec((1, H1 * C1), lambda i: (0, 0)),
          pl.BlockSpec((1, H1 * C1), lambda i: (0, 0)),
      ],
      out_specs=[
          pl.BlockSpec((BLK, RW1), lambda i: (i, 0)),
          pl.BlockSpec((BLK, 16), lambda i: (i, 0)),
          pl.BlockSpec((1, 8), lambda i: (0, 0)),
          pl.BlockSpec((1, 8), lambda i: (0, 0)),
      ],
      out_shape=[
          jax.ShapeDtypeStruct((NP, RW1), jnp.float32),
          jax.ShapeDtypeStruct((NP, 16), jnp.float32),
          jax.ShapeDtypeStruct((1, 8), jnp.float32),
          jax.ShapeDtypeStruct((1, 8), jnp.float32),
      ],
  )(x_pad, w1, as1, ad1)


def _ivec(v):
  return jnp.full((16,), v, jnp.int32)


def _make_edge_fn_l1(mv):
  base = lax.div(lax.broadcasted_iota(jnp.int32, (16,), 0), _ivec(8))
  idxs = [lax.add(base, _ivec(2 * k)) for k in range(4)]

  def edge(rows, adr, out, i):
    va = rows[i, pl.ds(0, 16)]
    vd = adr[i, pl.ds(0, 16)]
    al = va + vd
    e = jnp.exp(_leaky(al) - mv)
    out[i, pl.ds(0, 16)] = e
    for k in range(4):
      hk = rows[i, pl.ds(16 + 16 * k, 16)]
      mk = _vgather(e, idxs[k])
      out[i, pl.ds(16 + 16 * k, 16)] = hk * mk
  return edge


def _make_edge_fn_l2(mv):
  lane = lax.broadcasted_iota(jnp.int32, (16,), 0)
  eight = _ivec(8)
  msk = lane < eight

  def edge(rows, adr, out, i):
    r = rows[i, pl.ds(0, 16)]
    d = adr[i, pl.ds(0, 16)]
    al = r + d
    ef = jnp.exp(_leaky(al) - mv)
    g = _vgather(ef, eight)
    out[i, pl.ds(0, 16)] = jnp.where(msk, g * r, g)
  return edge


def _sc_edge_pass(rw, make_edge_fn, nbuf, inplace):
  mesh = plsc.VectorSubcoreMesh(core_axis_name="c", subcore_axis_name="s")

  def body(t_hbm, ad_hbm, src_hbm, dst_hbm, m_hbm, out_hbm,
           idx_s, idx_d, rows, adr, mvec, accum,
           sg_r, sg_a, ss, outb=None):
    c = lax.axis_index("c")
    s = lax.axis_index("s")
    wid = s * NC + c
    pltpu.sync_copy(src_hbm.at[wid], idx_s)
    pltpu.sync_copy(dst_hbm.at[wid], idx_d)
    pltpu.sync_copy(m_hbm, mvec)
    edge_fn = make_edge_fn(mvec[...])

    zb = rows.at[0]

    def zrow(r, carry):
      for k in range(rw // 16):
        zb[r, pl.ds(16 * k, 16)] = jnp.zeros((16,), jnp.float32)
      return carry
    lax.fori_loop(0, CH, zrow, 0)
    nfull, remr = ROWS_PER_TILE // CH, ROWS_PER_TILE % CH
    for k in range(nfull):
      pltpu.sync_copy(zb, accum.at[pl.ds(s * ROWS_PER_TILE + k * CH, CH)])
    if remr:
      pltpu.sync_copy(
          rows.at[0, pl.ds(0, remr)],
          accum.at[pl.ds(s * ROWS_PER_TILE + nfull * CH, remr)])
    plsc.subcore_barrier()

    def start_gathers(j, b):
      pltpu.async_copy(t_hbm.at[idx_s.at[j]], rows.at[b], sg_r.at[b])
      pltpu.async_copy(ad_hbm.at[idx_d.at[j]], adr.at[b], sg_a.at[b])

    def wait_gathers(j, b):
      pltpu.make_async_copy(t_hbm.at[idx_s.at[j]], rows.at[b],
                            sg_r.at[b]).wait()
      pltpu.make_async_copy(ad_hbm.at[idx_d.at[j]], adr.at[b],
                            sg_a.at[b]).wait()

    if inplace:
      def wait_scatter(j, b):
        pltpu.make_async_copy(rows.at[b], accum.at[idx_d.at[j]],
                              ss.at[b]).wait()

      for b in range(nbuf - 1):
        start_gathers(b, b)

      def outer(g, carry):
        for b in range(nbuf):
          j = g * nbuf + b
          wait_gathers(j, b)

          def edge2(i, c2):
            edge_fn(rows.at[b], adr.at[b], rows.at[b], i * 2)
            edge_fn(rows.at[b], adr.at[b], rows.at[b], i * 2 + 1)
            return c2
          lax.fori_loop(0, CH // 2, edge2, 0)
          pltpu.async_copy(rows.at[b], accum.at[idx_d.at[j]], ss.at[b],
                           add=True)
          bp = (b - 1) % nbuf

          @pl.when(j >= 1)
          def _():
            wait_scatter(j - 1, bp)

          @pl.when(j + nbuf - 1 < CPT)
          def _():
            start_gathers(j + nbuf - 1, bp)
        return carry
      lax.fori_loop(0, CPT // nbuf, outer, 0)
      wait_scatter(CPT - 1, (CPT - 1) % nbuf)
    else:
      def wait_scatter(j, b):
        pltpu.make_async_copy(outb.at[b], accum.at[idx_d.at[j]],
                              ss.at[b]).wait()

      for b in range(nbuf):
        start_gathers(b, b)

      def outer(g, carry):
        for b in range(nbuf):
          j = g * nbuf + b
          wait_gathers(j, b)
          ob = j % 2

          @pl.when(j >= 2)
          def _():
            wait_scatter(j - 2, ob)

          def edge2(i, c2):
            edge_fn(rows.at[b], adr.at[b], outb.at[ob], i * 2)
            edge_fn(rows.at[b], adr.at[b], outb.at[ob], i * 2 + 1)
            return c2
          lax.fori_loop(0, CH // 2, edge2, 0)
          pltpu.async_copy(outb.at[ob], accum.at[idx_d.at[j]], ss.at[ob],
                           add=True)

          @pl.when(j + nbuf < CPT)
          def _():
            start_gathers(j + nbuf, b)
        return carry
      lax.fori_loop(0, CPT // nbuf, outer, 0)
      wait_scatter(CPT - 2, (CPT - 2) % 2)
      wait_scatter(CPT - 1, (CPT - 1) % 2)
    plsc.subcore_barrier()
    for k in range(nfull):
      pltpu.sync_copy(accum.at[pl.ds(s * ROWS_PER_TILE + k * CH, CH)],
                      rows.at[0])
      pltpu.sync_copy(rows.at[0],
                      out_hbm.at[c, pl.ds(s * ROWS_PER_TILE + k * CH, CH)])
    if remr:
      off = s * ROWS_PER_TILE + nfull * CH
      pltpu.sync_copy(accum.at[pl.ds(off, remr)], rows.at[0, pl.ds(0, remr)])
      pltpu.sync_copy(rows.at[0, pl.ds(0, remr)],
                      out_hbm.at[c, pl.ds(off, remr)])

  scratch = [
      pltpu.VMEM((CPT, CH), jnp.int32),
      pltpu.VMEM((CPT, CH), jnp.int32),
      pltpu.VMEM((nbuf, CH, rw), jnp.float32),
      pltpu.VMEM((nbuf, CH, 16), jnp.float32),
      pltpu.VMEM((16,), jnp.float32),
      pltpu.VMEM_SHARED((NP, rw), jnp.float32),
      pltpu.SemaphoreType.DMA((nbuf,)),
      pltpu.SemaphoreType.DMA((nbuf,)),
      pltpu.SemaphoreType.DMA((nbuf,)),
  ]
  if not inplace:
    scratch.append(pltpu.VMEM((2, CH, rw), jnp.float32))
  kern = pl.kernel(
      body,
      out_type=jax.ShapeDtypeStruct((NC, NP, rw), jnp.float32),
      mesh=mesh,
      scratch_types=scratch,
      compiler_params=pltpu.CompilerParams(use_tc_tiling_on_sc=False),
  )
  return kern


def _stage_b_body(acc_ref, b1_ref, w2_ref, as2_ref, ad2_ref,
                  t2_ref, ad2o_ref, ms_ref, md_ref):
  i = pl.program_id(0)
  tot = acc_ref[0] + acc_ref[1]
  den = tot[:, 0:8]
  num = tot[:, 16:80]
  recip = 1.0 / (den + 1e-16)
  rr = lax.broadcasted_iota(jnp.int32, (H1, H1 * C1), 0)
  cc = lax.broadcasted_iota(jnp.int32, (H1, H1 * C1), 1)
  e8 = (cc // C1 == rr).astype(jnp.float32)
  out1 = num * jnp.dot(recip, e8, preferred_element_type=jnp.float32)
  out1 = out1 + b1_ref[...]
  x2 = jnp.where(out1 > 0, out1, jnp.exp(out1) - 1.0)
  h2 = jnp.dot(x2, w2_ref[...], preferred_element_type=jnp.float32)
  a_s2 = jnp.sum(h2 * as2_ref[...], axis=1, keepdims=True)
  a_d2 = jnp.sum(h2 * ad2_ref[...], axis=1, keepdims=True)
  blk = h2.shape[0]
  t2_ref[...] = jnp.concatenate(
      [h2, jnp.broadcast_to(a_s2, (blk, 8))], axis=1)
  ad2o_ref[...] = jnp.broadcast_to(a_d2, (blk, 16))
  cs = jnp.max(a_s2, axis=0, keepdims=True)
  cd = jnp.max(a_d2, axis=0, keepdims=True)

  @pl.when(i == 0)
  def _():
    ms_ref[...] = cs
    md_ref[...] = cd

  @pl.when(i > 0)
  def _():
    ms_ref[...] = jnp.maximum(ms_ref[...], cs)
    md_ref[...] = jnp.maximum(md_ref[...], cd)


def _stage_b(acc1, b1, w2, as2, ad2):
  return pl.pallas_call(
      _stage_b_body,
      grid=(NP // BLK,),
      in_specs=[
          pl.BlockSpec((NC, BLK, RW1), lambda i: (0, i, 0)),
          pl.BlockSpec((1, H1 * C1), lambda i: (0, 0)),
          pl.BlockSpec((H1 * C1, NCLS), lambda i: (0, 0)),
          pl.BlockSpec((1, NCLS), lambda i: (0, 0)),
          pl.BlockSpec((1, NCLS), lambda i: (0, 0)),
      ],
      out_specs=[
          pl.BlockSpec((BLK, RW2), lambda i: (i, 0)),
          pl.BlockSpec((BLK, 16), lambda i: (i, 0)),
          pl.BlockSpec((1, 1), lambda i: (0, 0)),
          pl.BlockSpec((1, 1), lambda i: (0, 0)),
      ],
      out_shape=[
          jax.ShapeDtypeStruct((NP, RW2), jnp.float32),
          jax.ShapeDtypeStruct((NP, 16), jnp.float32),
          jax.ShapeDtypeStruct((1, 1), jnp.float32),
          jax.ShapeDtypeStruct((1, 1), jnp.float32),
      ],
  )(acc1, b1, w2, as2, ad2)


def _stage_c_body(acc_ref, b2_ref, out_ref):
  tot = acc_ref[0] + acc_ref[1]
  num = tot[:, 0:8]
  den = tot[:, 8:9]
  o = num * (1.0 / (den + 1e-16)) + b2_ref[...]
  m = jnp.max(o, axis=1, keepdims=True)
  z = o - m
  lse = jnp.log(jnp.sum(jnp.exp(z), axis=1, keepdims=True))
  out_ref[...] = z - lse


def _stage_c(acc2, b2):
  return pl.pallas_call(
      _stage_c_body,
      grid=(NP // BLK,),
      in_specs=[
          pl.BlockSpec((NC, BLK, RW2), lambda i: (0, i, 0)),
          pl.BlockSpec((1, NCLS), lambda i: (0, 0)),
      ],
      out_specs=pl.BlockSpec((BLK, NCLS), lambda i: (i, 0)),
      out_shape=jax.ShapeDtypeStruct((NP, NCLS), jnp.float32),
  )(acc2, b2)


def kernel(x, edge_index, W1, att_src1, att_dst1, b1, W2, att_src2, att_dst2,
           b2):
  f32 = jnp.float32
  x_pad = jnp.pad(x, ((0, NP - N), (0, 0)))
  loop = jnp.arange(N, dtype=jnp.int32)
  npad = EPAD - E - N
  padidx = DUMMY + jnp.arange(npad, dtype=jnp.int32) % (NP - N)
  src = jnp.concatenate([edge_index[0].astype(jnp.int32), loop, padidx])
  dst = jnp.concatenate([edge_index[1].astype(jnp.int32), loop, padidx])
  src_r = src.reshape(NTILES, CPT, CH)
  dst_r = dst.reshape(NTILES, CPT, CH)

  as1 = att_src1.reshape(1, H1 * C1).astype(f32)
  ad1 = att_dst1.reshape(1, H1 * C1).astype(f32)
  t1, ad1t, ms1, md1 = _stage_a(x_pad, W1.astype(f32), as1, ad1)
  m1 = _leaky(ms1 + md1).reshape(H1)
  m1vec = jnp.concatenate([m1, jnp.zeros((8,), f32)])

  acc1 = _sc_edge_pass(RW1, _make_edge_fn_l1, 6, True)(
      t1, ad1t, src_r, dst_r, m1vec)

  as2 = att_src2.reshape(1, NCLS).astype(f32)
  ad2 = att_dst2.reshape(1, NCLS).astype(f32)
  t2, ad2t, ms2, md2 = _stage_b(acc1, b1.reshape(1, H1 * C1).astype(f32),
                                W2.astype(f32), as2, ad2)
  m2 = _leaky(ms2 + md2).reshape(())
  m2vec = jnp.full((16,), m2, f32)

  acc2 = _sc_edge_pass(RW2, _make_edge_fn_l2, 6, False)(
      t2, ad2t, src_r, dst_r, m2vec)

  out = _stage_c(acc2, b2.reshape(1, NCLS).astype(f32))
  return out[:N]
